# baseline (speedup 1.0000x reference)
import functools

import jax
import jax.numpy as jnp
from jax import lax
from jax.experimental import pallas as pl
from jax.experimental.pallas import tpu as pltpu
from jax.experimental.pallas import tpu_sc as plsc

F = 26
V = 1_000_000
VPAD = 1_000_448
B = 16384
NC = 2
NS = 16
NW = NC * NS
BPW = B // NW
N = BPW * F
LANES = 16
NCH = BPW // LANES

_mesh = plsc.VectorSubcoreMesh(core_axis_name="c", subcore_axis_name="s")


def _depad_body(w_ref, o_ref):
    o_ref[pl.ds(0, V)] = w_ref[0, 0, :]


_depad = pl.pallas_call(
    _depad_body,
    grid=(F,),
    in_specs=[pl.BlockSpec((1, 1, V), lambda f: (f, 0, 0))],
    out_specs=pl.BlockSpec((VPAD,), lambda f: (f,)),
    out_shape=jax.ShapeDtypeStruct((F * VPAD,), jnp.float32),
)


@functools.partial(
    pl.kernel,
    out_type=jax.ShapeDtypeStruct((B,), jnp.float32),
    mesh=_mesh,
    compiler_params=pltpu.CompilerParams(
        needs_layout_passes=False, use_tc_tiling_on_sc=False
    ),
    scratch_types=[
        pltpu.VMEM((N,), jnp.int32),
        pltpu.VMEM((N,), jnp.int32),
        pltpu.VMEM((N,), jnp.float32),
        pltpu.VMEM((BPW,), jnp.float32),
        pltpu.SemaphoreType.DMA,
    ],
)
def _linear_logits_sc(x_hbm, w_hbm, out_hbm, xblk, idxs, vals, accv, sem):
    wid = lax.axis_index("s") * NC + lax.axis_index("c")
    base = wid * BPW

    pltpu.sync_copy(x_hbm.at[pl.ds(base * F, N)], xblk)

    iota_f = lax.iota(jnp.int32, LANES) * F
    for f in range(F):
        def _build(j, _, f=f):
            pos = j * (LANES * F) + iota_f + f
            xv = plsc.load_gather(xblk, [pos])
            idxs[pl.ds(f * BPW + j * LANES, LANES)] = xv + f * VPAD
            return 0

        lax.fori_loop(0, NCH, _build, 0)

    pltpu.async_copy(w_hbm.at[idxs], vals, sem).wait()

    def _reduce(j, _):
        acc = vals[pl.ds(j * LANES, LANES)]
        for f in range(1, F):
            acc = acc + vals[pl.ds(f * BPW + j * LANES, LANES)]
        accv[pl.ds(j * LANES, LANES)] = acc
        return 0

    lax.fori_loop(0, NCH, _reduce, 0)

    pltpu.sync_copy(accv, out_hbm.at[pl.ds(base, BPW)])


def kernel(X, W):
    w_view = jnp.transpose(W, (0, 2, 1))
    w_flat = _depad(w_view)
    out = _linear_logits_sc(X.reshape(-1), w_flat)
    return out.reshape(B, 1)


# speedup vs baseline: 2.1648x; 2.1648x over previous
import functools

import jax
import jax.numpy as jnp
from jax import lax
from jax.experimental import pallas as pl
from jax.experimental.pallas import tpu as pltpu
from jax.experimental.pallas import tpu_sc as plsc

F = 26
V = 1_000_000
VALIGN = 999_936
B = 16384
NC = 2
NS = 16
NW = NC * NS
BPW = B // NW
N = BPW * F
LANES = 16
NCH = BPW // LANES

_mesh = plsc.VectorSubcoreMesh(core_axis_name="c", subcore_axis_name="s")


@functools.partial(
    pl.kernel,
    out_type=jax.ShapeDtypeStruct((B,), jnp.float32),
    mesh=_mesh,
    compiler_params=pltpu.CompilerParams(
        needs_layout_passes=False, use_tc_tiling_on_sc=True
    ),
    scratch_types=[
        pltpu.VMEM((N,), jnp.int32),
        pltpu.VMEM((N,), jnp.int32),
        pltpu.VMEM((N,), jnp.float32),
        pltpu.VMEM((BPW,), jnp.float32),
        pltpu.SemaphoreType.DMA,
    ],
)
def _linear_logits_sc(x_hbm, w_hbm, out_hbm, xblk, idxs, vals, accv, sem):
    wid = lax.axis_index("s") * NC + lax.axis_index("c")
    base = wid * BPW

    pltpu.sync_copy(x_hbm.at[pl.ds(base * F, N)], xblk)

    iota_f = lax.iota(jnp.int32, LANES) * F
    for f in range(F):
        def _build(j, _, f=f):
            pos = j * (LANES * F) + iota_f + f
            xv = plsc.load_gather(xblk, [pos])
            idxs[pl.ds(f * BPW + j * LANES, LANES)] = xv
            return 0

        lax.fori_loop(0, NCH, _build, 0)

    copies = [
        pltpu.async_copy(
            w_hbm.at[f, 0, pl.ds(0, VALIGN)].at[idxs.at[pl.ds(f * BPW, BPW)]],
            vals.at[pl.ds(f * BPW, BPW)],
            sem,
        )
        for f in range(F)
    ]
    for c in copies:
        c.wait()

    def _reduce(j, _):
        acc = vals[pl.ds(j * LANES, LANES)]
        for f in range(1, F):
            acc = acc + vals[pl.ds(f * BPW + j * LANES, LANES)]
        accv[pl.ds(j * LANES, LANES)] = acc
        return 0

    lax.fori_loop(0, NCH, _reduce, 0)

    pltpu.sync_copy(accv, out_hbm.at[pl.ds(base, BPW)])


def kernel(X, W):
    w_view = jnp.transpose(W, (0, 2, 1))
    out = _linear_logits_sc(X.reshape(-1), w_view)
    return out.reshape(B, 1)


# speedup vs baseline: 3.1015x; 1.4327x over previous
import functools

import jax
import jax.numpy as jnp
from jax import lax
from jax.experimental import pallas as pl
from jax.experimental.pallas import tpu as pltpu
from jax.experimental.pallas import tpu_sc as plsc

F = 26
V = 1_000_000
VALIGN = 999_936
B = 16384
NC = 2
NS = 16
NW = NC * NS
BPW = B // NW
N = BPW * F
LANES = 16
NCH = BPW // LANES

_mesh = plsc.VectorSubcoreMesh(core_axis_name="c", subcore_axis_name="s")


@functools.partial(
    pl.kernel,
    out_type=jax.ShapeDtypeStruct((B,), jnp.float32),
    mesh=_mesh,
    compiler_params=pltpu.CompilerParams(
        needs_layout_passes=False, use_tc_tiling_on_sc=True
    ),
    scratch_types=[
        pltpu.VMEM((N,), jnp.int32),
        pltpu.VMEM((N,), jnp.float32),
        pltpu.VMEM((BPW,), jnp.float32),
        pltpu.SemaphoreType.DMA,
    ],
)
def _linear_logits_sc(x_hbm, w_hbm, out_hbm, idxs, vals, accv, sem):
    wid = lax.axis_index("s") * NC + lax.axis_index("c")
    base = wid * BPW

    copies = []
    for f in range(F):
        seg = pl.ds(f * BPW, BPW)
        pltpu.sync_copy(x_hbm.at[f, pl.ds(base, BPW)], idxs.at[seg])
        copies.append(
            pltpu.async_copy(
                w_hbm.at[f, 0, pl.ds(0, VALIGN)].at[idxs.at[seg]],
                vals.at[seg],
                sem,
            )
        )
    for c in copies:
        c.wait()

    def _reduce(j, _):
        acc = vals[pl.ds(j * LANES, LANES)]
        for f in range(1, F):
            acc = acc + vals[pl.ds(f * BPW + j * LANES, LANES)]
        accv[pl.ds(j * LANES, LANES)] = acc
        return 0

    lax.fori_loop(0, NCH, _reduce, 0)

    pltpu.sync_copy(accv, out_hbm.at[pl.ds(base, BPW)])


def kernel(X, W):
    w_view = jnp.transpose(W, (0, 2, 1))
    x_view = X.T
    out = _linear_logits_sc(x_view, w_view)
    return out.reshape(B, 1)


# speedup vs baseline: 3.2177x; 1.0375x over previous
import functools

import jax
import jax.numpy as jnp
from jax import lax
from jax.experimental import pallas as pl
from jax.experimental.pallas import tpu as pltpu
from jax.experimental.pallas import tpu_sc as plsc

F = 26
V = 1_000_000
VALIGN = 999_936
B = 16384
NC = 2
NS = 16
NW = NC * NS
BPW = B // NW
N = BPW * F
LANES = 16
NCH = BPW // LANES

_mesh = plsc.VectorSubcoreMesh(core_axis_name="c", subcore_axis_name="s")


@functools.partial(
    pl.kernel,
    out_type=jax.ShapeDtypeStruct((B,), jnp.float32),
    mesh=_mesh,
    compiler_params=pltpu.CompilerParams(
        needs_layout_passes=False, use_tc_tiling_on_sc=True
    ),
    scratch_types=[
        pltpu.VMEM((N,), jnp.int32),
        pltpu.VMEM((N,), jnp.float32),
        pltpu.VMEM((BPW,), jnp.float32),
        pltpu.SemaphoreType.DMA,
        pltpu.SemaphoreType.DMA,
    ],
)
def _linear_logits_sc(x_hbm, w_hbm, out_hbm, idxs, vals, accv, sem, xsem):
    wid = lax.axis_index("s") * NC + lax.axis_index("c")
    base = wid * BPW

    idx_copies = [
        pltpu.async_copy(
            x_hbm.at[f, pl.ds(base, BPW)],
            idxs.at[pl.ds(f * BPW, BPW)],
            xsem,
        )
        for f in range(F)
    ]
    copies = []
    for f in range(F):
        seg = pl.ds(f * BPW, BPW)
        idx_copies[f].wait()
        copies.append(
            pltpu.async_copy(
                w_hbm.at[f, 0, pl.ds(0, VALIGN)].at[idxs.at[seg]],
                vals.at[seg],
                sem,
            )
        )
    for c in copies:
        c.wait()

    def _reduce(j, _):
        acc = vals[pl.ds(j * LANES, LANES)]
        for f in range(1, F):
            acc = acc + vals[pl.ds(f * BPW + j * LANES, LANES)]
        accv[pl.ds(j * LANES, LANES)] = acc
        return 0

    lax.fori_loop(0, NCH, _reduce, 0)

    pltpu.sync_copy(accv, out_hbm.at[pl.ds(base, BPW)])


def kernel(X, W):
    w_view = jnp.transpose(W, (0, 2, 1))
    x_view = X.T
    out = _linear_logits_sc(x_view, w_view)
    return out.reshape(B, 1)
